# baseline (speedup 1.0000x reference)
import functools

import jax
import jax.numpy as jnp
from jax import lax
from jax.experimental import pallas as pl
from jax.experimental.pallas import tpu as pltpu
from jax.experimental.pallas import tpu_sc as plsc

BATCH = 16384
D = 64
CR = 128


def kernel(user, item_i, item_j, embed_user, embed_item):
    info = plsc.get_sparse_core_info()
    NC, NS = info.num_cores, info.num_subcores
    NW = NC * NS
    BPW = BATCH // NW
    NCHK = BPW // CR

    u2 = user.reshape(NW, BPW)
    i2 = item_i.reshape(NW, BPW)
    j2 = item_j.reshape(NW, BPW)

    def pack(t):
        u = jax.lax.bitcast_convert_type(t, jnp.uint32)
        rnd = (u + jnp.uint32(0x7FFF) + ((u >> 16) & 1)) >> 16
        packed = rnd[:, 0::2] | (rnd[:, 1::2] << 16)
        return jax.lax.bitcast_convert_type(
            packed, jnp.int32).reshape(t.shape[0] // 4, 2 * D)

    eu32 = pack(embed_user)
    ei32 = pack(embed_item)

    mesh = plsc.VectorSubcoreMesh(core_axis_name="c", subcore_axis_name="s")

    @functools.partial(
        pl.kernel,
        out_type=(jax.ShapeDtypeStruct((BATCH,), jnp.float32),
                  jax.ShapeDtypeStruct((BATCH,), jnp.float32)),
        mesh=mesh,
        compiler_params=pltpu.CompilerParams(needs_layout_passes=False),
        scratch_types=[
            pltpu.VMEM((BPW,), jnp.int32),
            pltpu.VMEM((BPW,), jnp.int32),
            pltpu.VMEM((BPW,), jnp.int32),
            pltpu.VMEM((CR, 2 * D), jnp.int32),
            pltpu.VMEM((CR, 2 * D), jnp.int32),
            pltpu.VMEM((CR, 2 * D), jnp.int32),
            pltpu.VMEM((BPW,), jnp.float32),
            pltpu.VMEM((BPW,), jnp.float32),
            pltpu.SemaphoreType.DMA,
        ],
    )
    def bprmf(u_hbm, ii_hbm, ij_hbm, eu_hbm, ei_hbm, oi_hbm, oj_hbm,
              ru_v, ri_v, rj_v, gu_v, gi_v, gj_v, oi_v, oj_v, sem):
        wid = lax.axis_index("s") * NC + lax.axis_index("c")
        pltpu.sync_copy(u_hbm.at[wid], ru_v)
        pltpu.sync_copy(ii_hbm.at[wid], ri_v)
        pltpu.sync_copy(ij_hbm.at[wid], rj_v)

        iota16 = lax.iota(jnp.int32, 16)
        himask = jnp.full((16,), -65536, jnp.int32)

        def chunk_body(c, carry):
            def fire_body(g, carry2):
                base = pl.multiple_of(c * CR + g * 16, 16)
                uvec = ru_v[pl.ds(base, 16)] >> 2
                ivec = ri_v[pl.ds(base, 16)] >> 2
                jvec = rj_v[pl.ds(base, 16)] >> 2
                for l in range(16):
                    k = g * 16 + l
                    pltpu.async_copy(eu_hbm.at[uvec[l]], gu_v.at[k], sem)
                    pltpu.async_copy(ei_hbm.at[ivec[l]], gi_v.at[k], sem)
                    pltpu.async_copy(ei_hbm.at[jvec[l]], gj_v.at[k], sem)
                return carry2

            lax.fori_loop(0, CR // 16, fire_body, 0)
            pltpu.make_async_copy(eu_hbm.at[pl.ds(0, CR)], gu_v, sem).wait()
            pltpu.make_async_copy(eu_hbm.at[pl.ds(0, CR)], gi_v, sem).wait()
            pltpu.make_async_copy(eu_hbm.at[pl.ds(0, CR)], gj_v, sem).wait()

            def group_body(g, carry2):
                off = pl.multiple_of(c * CR + g * 16, 16)
                ur = ru_v[pl.ds(off, 16)]
                ir = ri_v[pl.ds(off, 16)]
                jr = rj_v[pl.ds(off, 16)]
                mu = (ur & 3) << 5
                mi = (ir & 3) << 5
                mj = (jr & 3) << 5
                items = g * 16 + iota16
                acc_i = jnp.zeros((16,), jnp.float32)
                acc_j = jnp.zeros((16,), jnp.float32)
                for p in range(D // 2):
                    pu = plsc.load_gather(gu_v, [items, mu + p])
                    pi = plsc.load_gather(gi_v, [items, mi + p])
                    pj = plsc.load_gather(gj_v, [items, mj + p])
                    ulo = plsc.bitcast(pu << 16, jnp.float32)
                    uhi = plsc.bitcast(pu & himask, jnp.float32)
                    ilo = plsc.bitcast(pi << 16, jnp.float32)
                    ihi = plsc.bitcast(pi & himask, jnp.float32)
                    jlo = plsc.bitcast(pj << 16, jnp.float32)
                    jhi = plsc.bitcast(pj & himask, jnp.float32)
                    acc_i = acc_i + ulo * ilo + uhi * ihi
                    acc_j = acc_j + ulo * jlo + uhi * jhi
                oi_v[pl.ds(off, 16)] = acc_i
                oj_v[pl.ds(off, 16)] = acc_j
                return carry2

            lax.fori_loop(0, CR // 16, group_body, 0)
            return carry

        lax.fori_loop(0, NCHK, chunk_body, 0)

        obase = pl.multiple_of(wid * BPW, BPW)
        pltpu.sync_copy(oi_v, oi_hbm.at[pl.ds(obase, BPW)])
        pltpu.sync_copy(oj_v, oj_hbm.at[pl.ds(obase, BPW)])

    return bprmf(u2, i2, j2, eu32, ei32)


# speedup vs baseline: 3.3876x; 3.3876x over previous
import functools

import jax
import jax.numpy as jnp
from jax import lax
from jax.experimental import pallas as pl
from jax.experimental.pallas import tpu as pltpu
from jax.experimental.pallas import tpu_sc as plsc

BATCH = 16384
D = 64
CR = 128


def kernel(user, item_i, item_j, embed_user, embed_item):
    info = plsc.get_sparse_core_info()
    NC, NS = info.num_cores, info.num_subcores
    NW = NC * NS
    BPW = BATCH // NW
    NCHK = BPW // CR

    u2 = user.reshape(NW, BPW)
    i2 = item_i.reshape(NW, BPW)
    j2 = item_j.reshape(NW, BPW)

    eu2 = embed_user.reshape(embed_user.shape[0] // 2, 2 * D)
    ei2 = embed_item.reshape(embed_item.shape[0] // 2, 2 * D)

    mesh = plsc.VectorSubcoreMesh(core_axis_name="c", subcore_axis_name="s")

    @functools.partial(
        pl.kernel,
        out_type=(jax.ShapeDtypeStruct((BATCH,), jnp.float32),
                  jax.ShapeDtypeStruct((BATCH,), jnp.float32)),
        mesh=mesh,
        compiler_params=pltpu.CompilerParams(needs_layout_passes=False),
        scratch_types=[
            pltpu.VMEM((BPW,), jnp.int32),
            pltpu.VMEM((BPW,), jnp.int32),
            pltpu.VMEM((BPW,), jnp.int32),
            pltpu.VMEM((CR, 2 * D), jnp.float32),
            pltpu.VMEM((CR, 2 * D), jnp.float32),
            pltpu.VMEM((CR, 2 * D), jnp.float32),
            pltpu.VMEM((BPW,), jnp.float32),
            pltpu.VMEM((BPW,), jnp.float32),
            pltpu.SemaphoreType.DMA,
        ],
    )
    def bprmf(u_hbm, ii_hbm, ij_hbm, eu_hbm, ei_hbm, oi_hbm, oj_hbm,
              ru_v, ri_v, rj_v, gu_v, gi_v, gj_v, oi_v, oj_v, sem):
        wid = lax.axis_index("s") * NC + lax.axis_index("c")
        pltpu.sync_copy(u_hbm.at[wid], ru_v)
        pltpu.sync_copy(ii_hbm.at[wid], ri_v)
        pltpu.sync_copy(ij_hbm.at[wid], rj_v)

        iota16 = lax.iota(jnp.int32, 16)

        def chunk_body(c, carry):
            def fire_body(g, carry2):
                base = pl.multiple_of(c * CR + g * 16, 16)
                uvec = ru_v[pl.ds(base, 16)] >> 1
                ivec = ri_v[pl.ds(base, 16)] >> 1
                jvec = rj_v[pl.ds(base, 16)] >> 1
                for l in range(16):
                    k = g * 16 + l
                    pltpu.async_copy(eu_hbm.at[uvec[l]], gu_v.at[k], sem)
                    pltpu.async_copy(ei_hbm.at[ivec[l]], gi_v.at[k], sem)
                    pltpu.async_copy(ei_hbm.at[jvec[l]], gj_v.at[k], sem)
                return carry2

            lax.fori_loop(0, CR // 16, fire_body, 0)
            pltpu.make_async_copy(eu_hbm.at[pl.ds(0, CR)], gu_v, sem).wait()
            pltpu.make_async_copy(eu_hbm.at[pl.ds(0, CR)], gi_v, sem).wait()
            pltpu.make_async_copy(eu_hbm.at[pl.ds(0, CR)], gj_v, sem).wait()

            def group_body(g, carry2):
                off = pl.multiple_of(c * CR + g * 16, 16)
                ur = ru_v[pl.ds(off, 16)]
                ir = ri_v[pl.ds(off, 16)]
                jr = rj_v[pl.ds(off, 16)]
                mu = (ur & 1) << 6
                mi = (ir & 1) << 6
                mj = (jr & 1) << 6
                items = g * 16 + iota16
                acc_i = jnp.zeros((16,), jnp.float32)
                acc_j = jnp.zeros((16,), jnp.float32)
                for d in range(D):
                    uu = plsc.load_gather(gu_v, [items, mu + d])
                    vi = plsc.load_gather(gi_v, [items, mi + d])
                    vj = plsc.load_gather(gj_v, [items, mj + d])
                    acc_i = acc_i + uu * vi
                    acc_j = acc_j + uu * vj
                oi_v[pl.ds(off, 16)] = acc_i
                oj_v[pl.ds(off, 16)] = acc_j
                return carry2

            lax.fori_loop(0, CR // 16, group_body, 0)
            return carry

        lax.fori_loop(0, NCHK, chunk_body, 0)

        obase = pl.multiple_of(wid * BPW, BPW)
        pltpu.sync_copy(oi_v, oi_hbm.at[pl.ds(obase, BPW)])
        pltpu.sync_copy(oj_v, oj_hbm.at[pl.ds(obase, BPW)])

    return bprmf(u2, i2, j2, eu2, ei2)


# speedup vs baseline: 3.6328x; 1.0724x over previous
import functools

import jax
import jax.numpy as jnp
from jax import lax
from jax.experimental import pallas as pl
from jax.experimental.pallas import tpu as pltpu
from jax.experimental.pallas import tpu_sc as plsc

BATCH = 16384
D = 64
CR = 128


def kernel(user, item_i, item_j, embed_user, embed_item):
    info = plsc.get_sparse_core_info()
    NC, NS = info.num_cores, info.num_subcores
    NW = NC * NS
    BPW = BATCH // NW
    NCHK = BPW // CR

    u2 = user.reshape(NW, BPW)
    i2 = item_i.reshape(NW, BPW)
    j2 = item_j.reshape(NW, BPW)

    eu2 = jax.lax.bitcast_convert_type(embed_user, jnp.int32)
    ei2 = jax.lax.bitcast_convert_type(embed_item, jnp.int32)

    mesh = plsc.VectorSubcoreMesh(core_axis_name="c", subcore_axis_name="s")

    @functools.partial(
        pl.kernel,
        out_type=(jax.ShapeDtypeStruct((BATCH,), jnp.float32),
                  jax.ShapeDtypeStruct((BATCH,), jnp.float32)),
        mesh=mesh,
        compiler_params=pltpu.CompilerParams(needs_layout_passes=False),
        scratch_types=[
            pltpu.VMEM((BPW,), jnp.int32),
            pltpu.VMEM((BPW,), jnp.int32),
            pltpu.VMEM((BPW,), jnp.int32),
            pltpu.VMEM((CR, D), jnp.int32),
            pltpu.VMEM((CR, D), jnp.int32),
            pltpu.VMEM((CR, D), jnp.int32),
            pltpu.VMEM((BPW,), jnp.float32),
            pltpu.VMEM((BPW,), jnp.float32),
            pltpu.SemaphoreType.DMA,
        ],
    )
    def bprmf(u_hbm, ii_hbm, ij_hbm, eu_hbm, ei_hbm, oi_hbm, oj_hbm,
              ru_v, ri_v, rj_v, gu_v, gi_v, gj_v, oi_v, oj_v, sem):
        wid = lax.axis_index("s") * NC + lax.axis_index("c")
        pltpu.sync_copy(u_hbm.at[wid], ru_v)
        pltpu.sync_copy(ii_hbm.at[wid], ri_v)
        pltpu.sync_copy(ij_hbm.at[wid], rj_v)

        iota16 = lax.iota(jnp.int32, 16)

        def chunk_body(c, carry):
            def fire_body(g, carry2):
                base = pl.multiple_of(c * CR + g * 16, 16)
                uvec = ru_v[pl.ds(base, 16)]
                ivec = ri_v[pl.ds(base, 16)]
                jvec = rj_v[pl.ds(base, 16)]
                for l in range(16):
                    k = g * 16 + l
                    pltpu.async_copy(eu_hbm.at[uvec[l]], gu_v.at[k], sem)
                    pltpu.async_copy(ei_hbm.at[ivec[l]], gi_v.at[k], sem)
                    pltpu.async_copy(ei_hbm.at[jvec[l]], gj_v.at[k], sem)
                return carry2

            lax.fori_loop(0, CR // 16, fire_body, 0)
            pltpu.make_async_copy(eu_hbm.at[pl.ds(0, CR)], gu_v, sem).wait()
            pltpu.make_async_copy(eu_hbm.at[pl.ds(0, CR)], gi_v, sem).wait()
            pltpu.make_async_copy(eu_hbm.at[pl.ds(0, CR)], gj_v, sem).wait()

            def group_body(g, carry2):
                off = pl.multiple_of(c * CR + g * 16, 16)
                ur = ru_v[pl.ds(off, 16)]
                ir = ri_v[pl.ds(off, 16)]
                jr = rj_v[pl.ds(off, 16)]
                items = g * 16 + iota16
                acc_i = jnp.zeros((16,), jnp.float32)
                acc_j = jnp.zeros((16,), jnp.float32)
                for d in range(D):
                    cols = jnp.full((16,), d, jnp.int32)
                    uu = plsc.bitcast(
                        plsc.load_gather(gu_v, [items, cols]), jnp.float32)
                    vi = plsc.bitcast(
                        plsc.load_gather(gi_v, [items, cols]), jnp.float32)
                    vj = plsc.bitcast(
                        plsc.load_gather(gj_v, [items, cols]), jnp.float32)
                    acc_i = acc_i + uu * vi
                    acc_j = acc_j + uu * vj
                oi_v[pl.ds(off, 16)] = acc_i
                oj_v[pl.ds(off, 16)] = acc_j
                return carry2

            lax.fori_loop(0, CR // 16, group_body, 0)
            return carry

        lax.fori_loop(0, NCHK, chunk_body, 0)

        obase = pl.multiple_of(wid * BPW, BPW)
        pltpu.sync_copy(oi_v, oi_hbm.at[pl.ds(obase, BPW)])
        pltpu.sync_copy(oj_v, oj_hbm.at[pl.ds(obase, BPW)])

    return bprmf(u2, i2, j2, eu2, ei2)


# speedup vs baseline: 5.1334x; 1.4130x over previous
import functools

import jax
import jax.numpy as jnp
from jax import lax
from jax.experimental import pallas as pl
from jax.experimental.pallas import tpu as pltpu
from jax.experimental.pallas import tpu_sc as plsc

BATCH = 16384
D = 64
CR = 128


def kernel(user, item_i, item_j, embed_user, embed_item):
    info = plsc.get_sparse_core_info()
    NC, NS = info.num_cores, info.num_subcores
    NW = NC * NS
    BPW = BATCH // NW
    NCHK = BPW // CR

    u2 = user.reshape(NW, BPW)
    i2 = item_i.reshape(NW, BPW)
    j2 = item_j.reshape(NW, BPW)

    mesh = plsc.VectorSubcoreMesh(core_axis_name="c", subcore_axis_name="s")

    @functools.partial(
        pl.kernel,
        out_type=(jax.ShapeDtypeStruct((BATCH,), jnp.float32),
                  jax.ShapeDtypeStruct((BATCH,), jnp.float32)),
        mesh=mesh,
        compiler_params=pltpu.CompilerParams(needs_layout_passes=False),
        scratch_types=[
            pltpu.VMEM((BPW,), jnp.int32),
            pltpu.VMEM((BPW,), jnp.int32),
            pltpu.VMEM((BPW,), jnp.int32),
            pltpu.VMEM((CR, D), jnp.float32),
            pltpu.VMEM((CR, D), jnp.float32),
            pltpu.VMEM((CR, D), jnp.float32),
            pltpu.VMEM((BPW,), jnp.float32),
            pltpu.VMEM((BPW,), jnp.float32),
            pltpu.SemaphoreType.DMA,
        ],
    )
    def bprmf(u_hbm, ii_hbm, ij_hbm, eu_hbm, ei_hbm, oi_hbm, oj_hbm,
              ru_v, ri_v, rj_v, gu_v, gi_v, gj_v, oi_v, oj_v, sem):
        wid = lax.axis_index("s") * NC + lax.axis_index("c")
        pltpu.sync_copy(u_hbm.at[wid], ru_v)
        pltpu.sync_copy(ii_hbm.at[wid], ri_v)
        pltpu.sync_copy(ij_hbm.at[wid], rj_v)

        iota16 = lax.iota(jnp.int32, 16)

        def chunk_body(c, carry):
            def fire_body(g, carry2):
                base = pl.multiple_of(c * CR + g * 16, 16)
                uvec = ru_v[pl.ds(base, 16)]
                ivec = ri_v[pl.ds(base, 16)]
                jvec = rj_v[pl.ds(base, 16)]
                for l in range(16):
                    k = g * 16 + l
                    pltpu.async_copy(eu_hbm.at[uvec[l]], gu_v.at[k], sem)
                    pltpu.async_copy(ei_hbm.at[ivec[l]], gi_v.at[k], sem)
                    pltpu.async_copy(ei_hbm.at[jvec[l]], gj_v.at[k], sem)
                return carry2

            lax.fori_loop(0, CR // 16, fire_body, 0)
            pltpu.make_async_copy(eu_hbm.at[pl.ds(0, CR)], gu_v, sem).wait()
            pltpu.make_async_copy(eu_hbm.at[pl.ds(0, CR)], gi_v, sem).wait()
            pltpu.make_async_copy(eu_hbm.at[pl.ds(0, CR)], gj_v, sem).wait()

            def group_body(g, carry2):
                items = g * 16 + iota16
                acc_i = jnp.zeros((16,), jnp.float32)
                acc_j = jnp.zeros((16,), jnp.float32)
                for d in range(D):
                    cols = jnp.full((16,), d, jnp.int32)
                    uu = plsc.load_gather(gu_v, [items, cols])
                    vi = plsc.load_gather(gi_v, [items, cols])
                    vj = plsc.load_gather(gj_v, [items, cols])
                    acc_i = acc_i + uu * vi
                    acc_j = acc_j + uu * vj
                off = pl.multiple_of(c * CR + g * 16, 16)
                oi_v[pl.ds(off, 16)] = acc_i
                oj_v[pl.ds(off, 16)] = acc_j
                return carry2

            lax.fori_loop(0, CR // 16, group_body, 0)
            return carry

        lax.fori_loop(0, NCHK, chunk_body, 0)

        obase = pl.multiple_of(wid * BPW, BPW)
        pltpu.sync_copy(oi_v, oi_hbm.at[pl.ds(obase, BPW)])
        pltpu.sync_copy(oj_v, oj_hbm.at[pl.ds(obase, BPW)])

    return bprmf(u2, i2, j2, embed_user, embed_item)
